# baseline (speedup 1.0000x reference)
import functools

import jax
import jax.numpy as jnp
from jax import lax
from jax.experimental import pallas as pl
from jax.experimental.pallas import tpu as pltpu
from jax.experimental.pallas import tpu_sc as plsc

V = 1000
D = 1000
N = 51200
VPAD = 1024
DPAD = 1024
NW = 32
BPW = N // NW
CH = 8
NCH = BPW // CH
NB = 10
LANES = 16


def _logz_body(tab_ref, out_ref):
    i = pl.program_id(0)
    x = tab_ref[...]
    m = jnp.max(x, axis=-1)
    s = jnp.sum(jnp.exp(x - m[:, None]), axis=-1)
    out_ref[i, :] = jnp.log(s) + m


def _compute_logz(tablep):
    out = pl.pallas_call(
        _logz_body,
        grid=(VPAD // 128,),
        in_specs=[pl.BlockSpec((128, DPAD), lambda i: (i, 0))],
        out_specs=pl.BlockSpec((VPAD // 128, 128), lambda i: (0, 0)),
        out_shape=jax.ShapeDtypeStruct((VPAD // 128, 128), jnp.float32),
    )(tablep)
    return out.reshape(VPAD)


def _sc_body(table_hbm, flatt_hbm, idx_hbm, fidx_hbm, logz_hbm,
             out_hbm, part_hbm,
             idx_v, fidx_v, picked_v, lz_v, rows_v, part_v, *sems):
    lsem = sems[0]
    gsem = sems[1:1 + NB]
    wsem = sems[1 + NB:1 + 2 * NB]
    wid = lax.axis_index("s") * 2 + lax.axis_index("c")
    base = wid * BPW
    pltpu.sync_copy(idx_hbm.at[pl.ds(base, BPW)], idx_v)
    pltpu.sync_copy(fidx_hbm.at[pl.ds(base, BPW)], fidx_v)

    def gather(c, b, sem):
        return pltpu.make_async_copy(
            table_hbm.at[idx_v.at[pl.ds(c * CH, CH)]], rows_v.at[b], sem)

    def write(c, b, sem):
        return pltpu.make_async_copy(
            rows_v.at[b], out_hbm.at[pl.ds(base + c * CH, CH)], sem)

    for b in range(NB):
        gather(b, b, gsem[b]).start()
    pick_dma = pltpu.make_async_copy(flatt_hbm.at[fidx_v], picked_v, lsem)
    pick_dma.start()
    lz_dma = pltpu.make_async_copy(logz_hbm.at[idx_v], lz_v, lsem)
    lz_dma.start()

    def outer(gg, carry):
        for b in range(NB):
            c = gg * NB + b
            gather(c, b, gsem[b]).wait()
            write(c, b, wsem[b]).start()

            @pl.when(c + NB < NCH)
            def _():
                write(c, b, wsem[b]).wait()
                gather(c + NB, b, gsem[b]).start()
        return carry

    lax.fori_loop(0, NCH // NB, outer, 0)
    for b in range(NB):
        write(NCH - NB + b, b, wsem[b]).wait()

    pick_dma.wait()
    lz_dma.wait()

    def lacc(k, acc):
        off = k * LANES
        return acc + (lz_v[pl.ds(off, LANES)] - picked_v[pl.ds(off, LANES)])

    acc = lax.fori_loop(0, BPW // LANES, lacc,
                        jnp.zeros((LANES,), jnp.float32))
    part_v[...] = acc
    pltpu.sync_copy(part_v, part_hbm.at[wid])


def _sc_call(tablez, flatt, idx_flat, fidx_flat, logz):
    mesh = plsc.VectorSubcoreMesh(core_axis_name="c", subcore_axis_name="s")
    f = functools.partial(
        pl.kernel,
        out_type=(jax.ShapeDtypeStruct((N, DPAD), jnp.float32),
                  jax.ShapeDtypeStruct((NW, LANES), jnp.float32)),
        mesh=mesh,
        scratch_types=[
            pltpu.VMEM((BPW,), jnp.int32),
            pltpu.VMEM((BPW,), jnp.int32),
            pltpu.VMEM((BPW,), jnp.float32),
            pltpu.VMEM((BPW,), jnp.float32),
            pltpu.VMEM((NB, CH, DPAD), jnp.float32),
            pltpu.VMEM((LANES,), jnp.float32),
        ] + [pltpu.SemaphoreType.DMA] * (1 + 2 * NB),
    )(_sc_body)
    return f(tablez, flatt, idx_flat, fidx_flat, logz)


def kernel(idx, targets, table):
    idx_flat = idx.reshape(N).astype(jnp.int32)
    tgt_flat = targets.reshape(N).astype(jnp.int32)
    fidx_flat = tgt_flat * V + idx_flat

    tablep = jnp.pad(table, ((0, VPAD - V), (0, DPAD - D)),
                     constant_values=float("-inf"))
    logz = _compute_logz(tablep)
    tablez = jnp.pad(table, ((0, 0), (0, DPAD - D)))
    flatt = table.T.reshape(V * D)

    outp, parts = _sc_call(tablez, flatt, idx_flat, fidx_flat, logz)

    logits2 = outp[:, :D]
    loss = jnp.sum(parts) / N
    return (logits2, loss)


# speedup vs baseline: 1.0020x; 1.0020x over previous
import functools

import jax
import jax.numpy as jnp
from jax import lax
from jax.experimental import pallas as pl
from jax.experimental.pallas import tpu as pltpu
from jax.experimental.pallas import tpu_sc as plsc

V = 1000
D = 1000
N = 51200
VPAD = 1024
DPAD = 1024
NW = 32
BPW = N // NW
CH = 16
NCH = BPW // CH
NB = 5
LANES = 16


def _logz_body(tab_ref, out_ref):
    i = pl.program_id(0)
    x = tab_ref[...]
    m = jnp.max(x, axis=-1)
    s = jnp.sum(jnp.exp(x - m[:, None]), axis=-1)
    out_ref[i, :] = jnp.log(s) + m


def _compute_logz(tablep):
    out = pl.pallas_call(
        _logz_body,
        grid=(VPAD // 128,),
        in_specs=[pl.BlockSpec((128, DPAD), lambda i: (i, 0))],
        out_specs=pl.BlockSpec((VPAD // 128, 128), lambda i: (0, 0)),
        out_shape=jax.ShapeDtypeStruct((VPAD // 128, 128), jnp.float32),
    )(tablep)
    return out.reshape(VPAD)


def _sc_body(table_hbm, flatt_hbm, idx_hbm, fidx_hbm, logz_hbm,
             out_hbm, part_hbm,
             idx_v, fidx_v, picked_v, lz_v, rows_v, part_v, *sems):
    lsem = sems[0]
    gsem = sems[1:1 + NB]
    wsem = sems[1 + NB:1 + 2 * NB]
    wid = lax.axis_index("s") * 2 + lax.axis_index("c")
    base = wid * BPW
    pltpu.sync_copy(idx_hbm.at[pl.ds(base, BPW)], idx_v)
    pltpu.sync_copy(fidx_hbm.at[pl.ds(base, BPW)], fidx_v)

    def gather(c, b, sem):
        return pltpu.make_async_copy(
            table_hbm.at[idx_v.at[pl.ds(c * CH, CH)]], rows_v.at[b], sem)

    def write(c, b, sem):
        return pltpu.make_async_copy(
            rows_v.at[b], out_hbm.at[pl.ds(base + c * CH, CH)], sem)

    for b in range(NB):
        gather(b, b, gsem[b]).start()
    pick_dma = pltpu.make_async_copy(flatt_hbm.at[fidx_v], picked_v, lsem)
    pick_dma.start()
    lz_dma = pltpu.make_async_copy(logz_hbm.at[idx_v], lz_v, lsem)
    lz_dma.start()

    def outer(gg, carry):
        for b in range(NB):
            c = gg * NB + b
            gather(c, b, gsem[b]).wait()
            write(c, b, wsem[b]).start()

            @pl.when(c + NB < NCH)
            def _():
                write(c, b, wsem[b]).wait()
                gather(c + NB, b, gsem[b]).start()
        return carry

    lax.fori_loop(0, NCH // NB, outer, 0)
    for b in range(NB):
        write(NCH - NB + b, b, wsem[b]).wait()

    pick_dma.wait()
    lz_dma.wait()

    def lacc(k, acc):
        off = k * LANES
        return acc + (lz_v[pl.ds(off, LANES)] - picked_v[pl.ds(off, LANES)])

    acc = lax.fori_loop(0, BPW // LANES, lacc,
                        jnp.zeros((LANES,), jnp.float32))
    part_v[...] = acc
    pltpu.sync_copy(part_v, part_hbm.at[wid])


def _sc_call(tablez, flatt, idx_flat, fidx_flat, logz):
    mesh = plsc.VectorSubcoreMesh(core_axis_name="c", subcore_axis_name="s")
    f = functools.partial(
        pl.kernel,
        out_type=(jax.ShapeDtypeStruct((N, DPAD), jnp.float32),
                  jax.ShapeDtypeStruct((NW, LANES), jnp.float32)),
        mesh=mesh,
        scratch_types=[
            pltpu.VMEM((BPW,), jnp.int32),
            pltpu.VMEM((BPW,), jnp.int32),
            pltpu.VMEM((BPW,), jnp.float32),
            pltpu.VMEM((BPW,), jnp.float32),
            pltpu.VMEM((NB, CH, DPAD), jnp.float32),
            pltpu.VMEM((LANES,), jnp.float32),
        ] + [pltpu.SemaphoreType.DMA] * (1 + 2 * NB),
    )(_sc_body)
    return f(tablez, flatt, idx_flat, fidx_flat, logz)


def kernel(idx, targets, table):
    idx_flat = idx.reshape(N).astype(jnp.int32)
    tgt_flat = targets.reshape(N).astype(jnp.int32)
    fidx_flat = tgt_flat * V + idx_flat

    tablep = jnp.pad(table, ((0, VPAD - V), (0, DPAD - D)),
                     constant_values=float("-inf"))
    logz = _compute_logz(tablep)
    tablez = jnp.pad(table, ((0, 0), (0, DPAD - D)))
    flatt = table.T.reshape(V * D)

    outp, parts = _sc_call(tablez, flatt, idx_flat, fidx_flat, logz)

    logits2 = outp[:, :D]
    loss = jnp.sum(parts) / N
    return (logits2, loss)


# speedup vs baseline: 1.0102x; 1.0082x over previous
import functools

import jax
import jax.numpy as jnp
from jax import lax
from jax.experimental import pallas as pl
from jax.experimental.pallas import tpu as pltpu
from jax.experimental.pallas import tpu_sc as plsc

V = 1000
D = 1000
N = 51200
VPAD = 1024
DPAD = 1024
NW = 32
BPW = N // NW
CH = 16
NCH = BPW // CH
NB = 5
LANES = 16


def _logz_body(tab_ref, out_ref):
    i = pl.program_id(0)
    x = tab_ref[...]
    m = jnp.max(x, axis=-1)
    s = jnp.sum(jnp.exp(x - m[:, None]), axis=-1)
    out_ref[i, :] = jnp.log(s) + m


def _compute_logz(tablep):
    out = pl.pallas_call(
        _logz_body,
        grid=(VPAD // 128,),
        in_specs=[pl.BlockSpec((128, DPAD), lambda i: (i, 0))],
        out_specs=pl.BlockSpec((VPAD // 128, 128), lambda i: (0, 0)),
        out_shape=jax.ShapeDtypeStruct((VPAD // 128, 128), jnp.float32),
    )(tablep)
    return out.reshape(VPAD)


def _sc_body(table_hbm, flatt_hbm, idx_hbm, fidx_hbm, logz_hbm,
             out_hbm, part_hbm,
             idx_v, fidx_v, picked_v, lz_v, rows_v, part_v, *sems):
    lsem = sems[0]
    gsem = sems[1:1 + NB]
    wsem = sems[1 + NB:1 + 2 * NB]
    wid = lax.axis_index("s") * 2 + lax.axis_index("c")
    base = wid * BPW
    pltpu.sync_copy(idx_hbm.at[pl.ds(base, BPW)], idx_v)
    pltpu.sync_copy(fidx_hbm.at[pl.ds(base, BPW)], fidx_v)

    def gather(c, b, sem):
        return pltpu.make_async_copy(
            table_hbm.at[idx_v.at[pl.ds(c * CH, CH)]], rows_v.at[b], sem)

    def write(c, b, sem):
        return pltpu.make_async_copy(
            rows_v.at[b], out_hbm.at[pl.ds(base + c * CH, CH)], sem)

    for b in range(NB):
        gather(b, b, gsem[b]).start()
    pick_dma = pltpu.make_async_copy(flatt_hbm.at[fidx_v], picked_v, lsem)
    pick_dma.start()
    lz_dma = pltpu.make_async_copy(logz_hbm.at[idx_v], lz_v, lsem)
    lz_dma.start()

    def outer(gg, carry):
        for b in range(NB):
            c = gg * NB + b
            gather(c, b, gsem[b]).wait()
            write(c, b, wsem[b]).start()

            @pl.when(c + NB < NCH)
            def _():
                write(c, b, wsem[b]).wait()
                gather(c + NB, b, gsem[b]).start()
        return carry

    lax.fori_loop(0, NCH // NB, outer, 0)
    for b in range(NB):
        write(NCH - NB + b, b, wsem[b]).wait()

    pick_dma.wait()
    lz_dma.wait()

    def lacc(k, acc):
        off = k * LANES
        return acc + (lz_v[pl.ds(off, LANES)] - picked_v[pl.ds(off, LANES)])

    acc = lax.fori_loop(0, BPW // LANES, lacc,
                        jnp.zeros((LANES,), jnp.float32))
    part_v[...] = acc
    pltpu.sync_copy(part_v, part_hbm.at[wid])


def _sc_call(tablez, flatt, idx_flat, fidx_flat, logz):
    mesh = plsc.VectorSubcoreMesh(core_axis_name="c", subcore_axis_name="s")
    f = functools.partial(
        pl.kernel,
        out_type=(jax.ShapeDtypeStruct((N, DPAD), jnp.float32),
                  jax.ShapeDtypeStruct((NW, LANES), jnp.float32)),
        mesh=mesh,
        scratch_types=[
            pltpu.VMEM((BPW,), jnp.int32),
            pltpu.VMEM((BPW,), jnp.int32),
            pltpu.VMEM((BPW,), jnp.float32),
            pltpu.VMEM((BPW,), jnp.float32),
            pltpu.VMEM((NB, CH, DPAD), jnp.float32),
            pltpu.VMEM((LANES,), jnp.float32),
        ] + [pltpu.SemaphoreType.DMA] * (1 + 2 * NB),
    )(_sc_body)
    return f(tablez, flatt, idx_flat, fidx_flat, logz)


def kernel(idx, targets, table):
    idx_flat = idx.reshape(N).astype(jnp.int32)
    tgt_flat = targets.reshape(N).astype(jnp.int32)
    fidx_flat = tgt_flat * V + idx_flat

    tablep = jnp.pad(table, ((0, VPAD - V), (0, DPAD - D)),
                     constant_values=float("-inf"))
    logz = _compute_logz(tablep)
    flatt = table.T.reshape(V * D)

    outp, parts = _sc_call(tablep, flatt, idx_flat, fidx_flat, logz)

    logits2 = outp[:, :D]
    loss = jnp.sum(parts) / N
    return (logits2, loss)
